# initial kernel scaffold (unmeasured)
import jax
import jax.numpy as jnp
from jax import lax
from jax.experimental import pallas as pl
from jax.experimental.pallas import tpu as pltpu


def kernel(
    x,
):
    def body(*refs):
        pass

    out_shape = jax.ShapeDtypeStruct(..., jnp.float32)
    return pl.pallas_call(body, out_shape=out_shape)(...)


# baseline (device time: 402229 ns/iter reference)
import jax
import jax.numpy as jnp
from jax import lax
from jax.experimental import pallas as pl
from jax.experimental.pallas import tpu as pltpu


def kernel(x):
    m, n = x.shape

    def body(x_hbm, recv_hbm, send_sem, recv_sem):
        my_x = lax.axis_index("x")
        my_y = lax.axis_index("y")
        my_z = lax.axis_index("z")

        rdma = pltpu.make_async_remote_copy(
            src_ref=x_hbm,
            dst_ref=recv_hbm,
            send_sem=send_sem,
            recv_sem=recv_sem,
            device_id=(my_x, my_y, 1 - my_z),
            device_id_type=pl.DeviceIdType.MESH,
        )
        rdma.start()
        rdma.wait()

    recv = pl.pallas_call(
        body,
        out_shape=jax.ShapeDtypeStruct((m, n), x.dtype),
        in_specs=[pl.BlockSpec(memory_space=pl.ANY)],
        out_specs=pl.BlockSpec(memory_space=pl.ANY),
        scratch_shapes=[
            pltpu.SemaphoreType.DMA,
            pltpu.SemaphoreType.DMA,
        ],
    )(x)
    return x + recv


# device time: 306742 ns/iter; 1.3113x vs baseline; 1.3113x over previous
import jax
import jax.numpy as jnp
from jax import lax
from jax.experimental import pallas as pl
from jax.experimental.pallas import tpu as pltpu

MESH = pl.DeviceIdType.MESH


def kernel(x):
    m, n = x.shape
    Q = m // 4

    def body(x_hbm, out_hbm, zrecv, xq, red, zsend_sem, zrecv_sem,
             copy_sems, psend_sems, precv_sems):
        mx = lax.axis_index("x")
        my = lax.axis_index("y")
        mz = lax.axis_index("z")
        q = 2 * mx + my
        row0 = q * Q

        zr = pltpu.make_async_remote_copy(
            src_ref=x_hbm.at[pl.ds(row0, Q)],
            dst_ref=zrecv,
            send_sem=zsend_sem,
            recv_sem=zrecv_sem,
            device_id=(mx, my, 1 - mz),
            device_id_type=MESH,
        )
        zr.start()

        cp_in = pltpu.make_async_copy(
            x_hbm.at[pl.ds(row0, Q)], xq, copy_sems.at[0]
        )
        cp_in.start()
        cp_in.wait()
        zr.wait()

        red[...] = xq[...] + zrecv[...]

        cp_out = pltpu.make_async_copy(
            red, out_hbm.at[pl.ds(row0, Q)], copy_sems.at[1]
        )
        cp_out.start()

        targets = [
            (1 - mx, my, mz),
            (mx, 1 - my, mz),
            (1 - mx, 1 - my, mz),
        ]
        sends = []
        for k, tgt in enumerate(targets):
            r = pltpu.make_async_remote_copy(
                src_ref=red,
                dst_ref=out_hbm.at[pl.ds(row0, Q)],
                send_sem=psend_sems.at[k],
                recv_sem=precv_sems.at[k],
                device_id=tgt,
                device_id_type=MESH,
            )
            r.start()
            sends.append(r)

        src_quarters = [
            2 * (1 - mx) + my,
            2 * mx + (1 - my),
            2 * (1 - mx) + (1 - my),
        ]
        for k, qs in enumerate(src_quarters):
            rwait = pltpu.make_async_remote_copy(
                src_ref=red,
                dst_ref=out_hbm.at[pl.ds(qs * Q, Q)],
                send_sem=psend_sems.at[k],
                recv_sem=precv_sems.at[k],
                device_id=targets[k],
                device_id_type=MESH,
            )
            rwait.wait_recv()

        for r in sends:
            r.wait_send()
        cp_out.wait()

    out = pl.pallas_call(
        body,
        out_shape=jax.ShapeDtypeStruct((m, n), x.dtype),
        in_specs=[pl.BlockSpec(memory_space=pl.ANY)],
        out_specs=pl.BlockSpec(memory_space=pl.ANY),
        scratch_shapes=[
            pltpu.VMEM((Q, n), x.dtype),
            pltpu.VMEM((Q, n), x.dtype),
            pltpu.VMEM((Q, n), x.dtype),
            pltpu.SemaphoreType.DMA,
            pltpu.SemaphoreType.DMA,
            pltpu.SemaphoreType.DMA((2,)),
            pltpu.SemaphoreType.DMA((3,)),
            pltpu.SemaphoreType.DMA((3,)),
        ],
    )(x)
    return out


# device time: 193060 ns/iter; 2.0834x vs baseline; 1.5888x over previous
import jax
import jax.numpy as jnp
from jax import lax
from jax.experimental import pallas as pl
from jax.experimental.pallas import tpu as pltpu

MESH = pl.DeviceIdType.MESH

CH = 512
N_ZCH = 6
N_PCH = 10


def kernel(x):
    m, n = x.shape
    Q = m // 4

    def body(x_hbm, out_hbm, zrecv, xl, red,
             zsend_sems, zrecv_sems, psend_sems, precv_sems, lcopy_sems):
        mx = lax.axis_index("x")
        my = lax.axis_index("y")
        mz = lax.axis_index("z")
        q = 2 * mx + my
        p = 2 * (1 - mx) + (1 - my)
        qx = 2 * (1 - mx) + my
        qy = 2 * mx + (1 - my)

        x_nbr = (1 - mx, my, mz)
        y_nbr = (mx, 1 - my, mz)
        diag = (1 - mx, 1 - my, mz)
        z_par = (mx, my, 1 - mz)

        zchunks = [(i * CH, q * Q + i * CH) for i in range(4)] + [
            (2048 + j * CH, p * Q + 1024 + j * CH) for j in range(2)
        ]

        zs = []
        for i, (o, g) in enumerate(zchunks):
            r = pltpu.make_async_remote_copy(
                src_ref=x_hbm.at[pl.ds(g, CH)],
                dst_ref=zrecv.at[pl.ds(o, CH)],
                send_sem=zsend_sems.at[i],
                recv_sem=zrecv_sems.at[i],
                device_id=z_par,
                device_id_type=MESH,
            )
            r.start()
            zs.append(r)

        cp_q = pltpu.make_async_copy(
            x_hbm.at[pl.ds(q * Q, Q)], xl.at[pl.ds(0, Q)], lcopy_sems.at[6]
        )
        cp_q.start()
        cp_p = pltpu.make_async_copy(
            x_hbm.at[pl.ds(p * Q + 1024, 2 * CH)],
            xl.at[pl.ds(2048, 2 * CH)],
            lcopy_sems.at[7],
        )
        cp_p.start()
        cp_q.wait()
        cp_p.wait()

        deps = {
            0: [(0, 0, q * Q, x_nbr), (3, 0, q * Q, y_nbr),
                (6, 0, q * Q, diag)],
            1: [(1, CH, q * Q + CH, x_nbr), (4, CH, q * Q + CH, y_nbr),
                (7, CH, q * Q + CH, diag)],
            2: [(2, 2 * CH, q * Q + 2 * CH, x_nbr)],
            3: [(5, 3 * CH, q * Q + 3 * CH, y_nbr)],
            4: [(8, 2048, p * Q + 1024, x_nbr)],
            5: [(9, 2048 + CH, p * Q + 1024 + CH, y_nbr)],
        }

        sends = []
        local_cps = []
        for i, (o, g) in enumerate(zchunks):
            zs[i].wait_recv()
            red[o:o + CH, :] = xl[o:o + CH, :] + zrecv[o:o + CH, :]
            cp = pltpu.make_async_copy(
                red.at[pl.ds(o, CH)], out_hbm.at[pl.ds(g, CH)],
                lcopy_sems.at[i],
            )
            cp.start()
            local_cps.append(cp)
            for k, o2, g2, tgt in deps[i]:
                r = pltpu.make_async_remote_copy(
                    src_ref=red.at[pl.ds(o2, CH)],
                    dst_ref=out_hbm.at[pl.ds(g2, CH)],
                    send_sem=psend_sems.at[k],
                    recv_sem=precv_sems.at[k],
                    device_id=tgt,
                    device_id_type=MESH,
                )
                r.start()
                sends.append(r)

        incoming = [
            (0, qx * Q, x_nbr),
            (1, qx * Q + CH, x_nbr),
            (2, qx * Q + 2 * CH, x_nbr),
            (3, qy * Q, y_nbr),
            (4, qy * Q + CH, y_nbr),
            (5, qy * Q + 3 * CH, y_nbr),
            (6, p * Q, diag),
            (7, p * Q + CH, diag),
            (8, qy * Q + 2 * CH, x_nbr),
            (9, qx * Q + 3 * CH, y_nbr),
        ]
        for k, g, src in incoming:
            rwait = pltpu.make_async_remote_copy(
                src_ref=red.at[pl.ds(0, CH)],
                dst_ref=out_hbm.at[pl.ds(g, CH)],
                send_sem=psend_sems.at[k],
                recv_sem=precv_sems.at[k],
                device_id=src,
                device_id_type=MESH,
            )
            rwait.wait_recv()

        for r in zs:
            r.wait_send()
        for r in sends:
            r.wait_send()
        for cp in local_cps:
            cp.wait()

    out = pl.pallas_call(
        body,
        out_shape=jax.ShapeDtypeStruct((m, n), x.dtype),
        in_specs=[pl.BlockSpec(memory_space=pl.ANY)],
        out_specs=pl.BlockSpec(memory_space=pl.ANY),
        scratch_shapes=[
            pltpu.VMEM((3072, n), x.dtype),
            pltpu.VMEM((3072, n), x.dtype),
            pltpu.VMEM((3072, n), x.dtype),
            pltpu.SemaphoreType.DMA((N_ZCH,)),
            pltpu.SemaphoreType.DMA((N_ZCH,)),
            pltpu.SemaphoreType.DMA((N_PCH,)),
            pltpu.SemaphoreType.DMA((N_PCH,)),
            pltpu.SemaphoreType.DMA((8,)),
        ],
        compiler_params=pltpu.CompilerParams(
            vmem_limit_bytes=64 * 1024 * 1024,
        ),
    )(x)
    return out


# device time: 182074 ns/iter; 2.2092x vs baseline; 1.0603x over previous
import jax
import jax.numpy as jnp
from jax import lax
from jax.experimental import pallas as pl
from jax.experimental.pallas import tpu as pltpu

MESH = pl.DeviceIdType.MESH

CH = 256
N_ZCH = 12
N_PCH = 20


def kernel(x):
    m, n = x.shape
    Q = m // 4

    def body(x_hbm, out_hbm, zrecv, xl, red,
             zsend_sems, zrecv_sems, psend_sems, precv_sems, lcopy_sems):
        mx = lax.axis_index("x")
        my = lax.axis_index("y")
        mz = lax.axis_index("z")
        q = 2 * mx + my
        p = 2 * (1 - mx) + (1 - my)
        qx = 2 * (1 - mx) + my
        qy = 2 * mx + (1 - my)

        x_nbr = (1 - mx, my, mz)
        y_nbr = (mx, 1 - my, mz)
        diag = (1 - mx, 1 - my, mz)
        z_par = (mx, my, 1 - mz)

        zchunks = [(i * CH, q * Q + i * CH) for i in range(8)] + [
            (2048 + j * CH, p * Q + 1024 + j * CH) for j in range(4)
        ]

        zs = []
        for i, (o, g) in enumerate(zchunks):
            r = pltpu.make_async_remote_copy(
                src_ref=x_hbm.at[pl.ds(g, CH)],
                dst_ref=zrecv.at[pl.ds(o, CH)],
                send_sem=zsend_sems.at[i],
                recv_sem=zrecv_sems.at[i],
                device_id=z_par,
                device_id_type=MESH,
            )
            r.start()
            zs.append(r)

        cp_q = pltpu.make_async_copy(
            x_hbm.at[pl.ds(q * Q, Q)], xl.at[pl.ds(0, Q)], lcopy_sems.at[12]
        )
        cp_q.start()
        cp_p = pltpu.make_async_copy(
            x_hbm.at[pl.ds(p * Q + 1024, 4 * CH)],
            xl.at[pl.ds(2048, 4 * CH)],
            lcopy_sems.at[13],
        )
        cp_p.start()
        cp_q.wait()
        cp_p.wait()

        deps = {i: [] for i in range(N_ZCH)}
        for i in range(6):
            deps[i].append((i, i * CH, q * Q + i * CH, x_nbr))
        for i in range(4):
            deps[i].append((6 + i, i * CH, q * Q + i * CH, y_nbr))
        for i in range(2):
            deps[6 + i].append(
                (10 + i, (6 + i) * CH, q * Q + (6 + i) * CH, y_nbr))
        for i in range(4):
            deps[i].append((12 + i, i * CH, q * Q + i * CH, diag))
        for i in range(2):
            deps[8 + i].append(
                (16 + i, 2048 + i * CH, p * Q + 1024 + i * CH, x_nbr))
        for i in range(2):
            deps[10 + i].append(
                (18 + i, 2560 + i * CH, p * Q + 1536 + i * CH, y_nbr))

        sends = []
        local_cps = []
        for i, (o, g) in enumerate(zchunks):
            zs[i].wait_recv()
            red[o:o + CH, :] = xl[o:o + CH, :] + zrecv[o:o + CH, :]
            cp = pltpu.make_async_copy(
                red.at[pl.ds(o, CH)], out_hbm.at[pl.ds(g, CH)],
                lcopy_sems.at[i],
            )
            cp.start()
            local_cps.append(cp)
            for k, o2, g2, tgt in deps[i]:
                r = pltpu.make_async_remote_copy(
                    src_ref=red.at[pl.ds(o2, CH)],
                    dst_ref=out_hbm.at[pl.ds(g2, CH)],
                    send_sem=psend_sems.at[k],
                    recv_sem=precv_sems.at[k],
                    device_id=tgt,
                    device_id_type=MESH,
                )
                r.start()
                sends.append(r)

        incoming = (
            [(i, qx * Q + i * CH, x_nbr) for i in range(6)]
            + [(6 + i, qy * Q + i * CH, y_nbr) for i in range(4)]
            + [(10 + i, qy * Q + (6 + i) * CH, y_nbr) for i in range(2)]
            + [(12 + i, p * Q + i * CH, diag) for i in range(4)]
            + [(16 + i, qy * Q + 1024 + i * CH, x_nbr) for i in range(2)]
            + [(18 + i, qx * Q + 1536 + i * CH, y_nbr) for i in range(2)]
        )
        for k, g, src in incoming:
            rwait = pltpu.make_async_remote_copy(
                src_ref=red.at[pl.ds(0, CH)],
                dst_ref=out_hbm.at[pl.ds(g, CH)],
                send_sem=psend_sems.at[k],
                recv_sem=precv_sems.at[k],
                device_id=src,
                device_id_type=MESH,
            )
            rwait.wait_recv()

        for r in zs:
            r.wait_send()
        for r in sends:
            r.wait_send()
        for cp in local_cps:
            cp.wait()

    out = pl.pallas_call(
        body,
        out_shape=jax.ShapeDtypeStruct((m, n), x.dtype),
        in_specs=[pl.BlockSpec(memory_space=pl.ANY)],
        out_specs=pl.BlockSpec(memory_space=pl.ANY),
        scratch_shapes=[
            pltpu.VMEM((3072, n), x.dtype),
            pltpu.VMEM((3072, n), x.dtype),
            pltpu.VMEM((3072, n), x.dtype),
            pltpu.SemaphoreType.DMA((N_ZCH,)),
            pltpu.SemaphoreType.DMA((N_ZCH,)),
            pltpu.SemaphoreType.DMA((N_PCH,)),
            pltpu.SemaphoreType.DMA((N_PCH,)),
            pltpu.SemaphoreType.DMA((14,)),
        ],
        compiler_params=pltpu.CompilerParams(
            vmem_limit_bytes=64 * 1024 * 1024,
        ),
    )(x)
    return out
